# baseline (speedup 1.0000x reference)
import functools

import jax
import jax.numpy as jnp
from jax import lax
from jax.experimental import pallas as pl
from jax.experimental.pallas import tpu as pltpu
from jax.experimental.pallas import tpu_sc as plsc

_N = 10
_H = 512
_E = 160
_K = 3
_NP = 16


def _edge_counts_sc(edge_ref, out_ref, src_v, dst_v, idx_v, ones_v, zeros_v,
                    acc):
    @pl.when((lax.axis_index("c") == 0) & (lax.axis_index("s") == 0))
    def _():
        pltpu.sync_copy(edge_ref.at[0], src_v)
        pltpu.sync_copy(edge_ref.at[1], dst_v)
        zrow = jnp.zeros((_NP,), jnp.float32)
        for r in range(_NP * _NP):
            zeros_v[r, :] = zrow
        onerow = jnp.full((_NP,), 1.0, jnp.float32)
        for e in range(_E):
            ones_v[e, :] = onerow
        for c in range(_E // _NP):
            s_ids = src_v[pl.ds(c * _NP, _NP)]
            d_ids = dst_v[pl.ds(c * _NP, _NP)]
            idx_v[pl.ds(c * _NP, _NP)] = d_ids * _NP + s_ids
        pltpu.sync_copy(zeros_v, acc)
        pltpu.sync_copy(ones_v, acc.at[idx_v], add=True)
        pltpu.sync_copy(acc, out_ref)


_edge_counts = pl.kernel(
    _edge_counts_sc,
    mesh=plsc.VectorSubcoreMesh(core_axis_name="c", subcore_axis_name="s"),
    out_type=jax.ShapeDtypeStruct((_NP * _NP, _NP), jnp.float32),
    scratch_types=[pltpu.VMEM((_E,), jnp.int32),
                   pltpu.VMEM((_E,), jnp.int32),
                   pltpu.VMEM((_E,), jnp.int32),
                   pltpu.VMEM((_E, _NP), jnp.float32),
                   pltpu.VMEM((_NP * _NP, _NP), jnp.float32),
                   pltpu.VMEM_SHARED((_NP * _NP, _NP), jnp.float32)],
)


def _fused(x_ref, a_ref,
           b10, b20, b11, b21, b12, b22, b13, b23,
           sa_w1_ref, sa_w2_ref, sa_w3_ref, bs_ref, vs_wT_ref,
           fc_w_ref, fc_b_ref, gcn_w_ref, gcn_b_ref,
           w10, w20, w11, w21, w12, w22, w13, w23,
           bp_ref, bm_ref):
    f32 = jnp.float32
    w_refs = (w10, w20, w11, w21, w12, w22, w13, w23)
    b_refs = (b10, b20, b11, b21, b12, b22, b13, b23)

    X = x_ref[...]
    for i in range(8):
        out = jax.lax.dot_general(X, w_refs[i][...], (((1,), (1,)), ((), ())),
                                  preferred_element_type=f32)
        out = out + b_refs[i][...]
        if i % 2 == 0:
            X_res, X = X, jnp.maximum(out, 0.0)
        else:
            X = jnp.maximum(jnp.maximum(out, 0.0) + X_res, 0.0)
    emb = X

    left = jnp.sum(emb * sa_w2_ref[...], axis=1, keepdims=True) * sa_w1_ref[0, 0]
    right = jnp.sum(emb * sa_w3_ref[...], axis=1, keepdims=True)
    sa_x = jax.lax.dot_general(left, right, (((1,), (1,)), ((), ())),
                               preferred_element_type=f32)
    sa_x = sa_x + bs_ref[...]
    sa_s = jnp.dot(jax.nn.sigmoid(sa_x), vs_wT_ref[...],
                   preferred_element_type=f32)

    A = a_ref[...][:_N, :_N]
    deg_in = jnp.sum(A, axis=1, keepdims=True)
    deg_out = jnp.sum(A, axis=0, keepdims=True)
    norm_in = jnp.where(deg_in > 0,
                        jax.lax.rsqrt(jnp.maximum(deg_in, 1e-12)), 0.0)
    norm_out = jnp.where(deg_out > 0,
                         jax.lax.rsqrt(jnp.maximum(deg_out, 1e-12)), 0.0)
    M = A * norm_in * norm_out
    agg = jnp.dot(M, emb, preferred_element_type=f32)
    g_emb = jnp.dot(agg, gcn_w_ref[...], preferred_element_type=f32) + gcn_b_ref[...]

    sa_ag = jnp.dot(sa_s, g_emb, preferred_element_type=f32)
    logits = jnp.sum(sa_ag * fc_w_ref[...], axis=1, keepdims=True) + fc_b_ref[0, 0]
    scores = jax.nn.sigmoid(logits)

    e1 = jnp.exp(scores)
    bp_ref[...] = e1 / jnp.sum(e1)
    e2 = jnp.exp(1.0 - scores)
    bm_ref[...] = e2 / jnp.sum(e2)


@jax.jit
def kernel(x, edge_index, tcn_params, sa_w1, sa_w2, sa_w3, bs, vs_w,
           fc_w, fc_b, gcn_w, gcn_b):
    A16 = _edge_counts(edge_index)[:, 0].reshape(_NP, _NP)

    vmem_ins = [x[:, :, 0], A16]
    vmem_ins += [b[None, :] for (w1, b1, w2, b2) in tcn_params for b in (b1, b2)]
    vmem_ins += [sa_w1, sa_w2.T, sa_w3, bs[None, :], vs_w.T,
                 fc_w, fc_b[None, :], gcn_w, gcn_b[None, :]]
    w_ins = [w[:, :, _K - 1] for (w1, b1, w2, b2) in tcn_params
             for w in (w1, w2)]

    bp, bm = pl.pallas_call(
        _fused,
        out_shape=[jax.ShapeDtypeStruct((_N, 1), jnp.float32),
                   jax.ShapeDtypeStruct((_N, 1), jnp.float32)],
    )(*vmem_ins, *w_ins)
    return bp[:, 0], bm[:, 0]


# speedup vs baseline: 1.0375x; 1.0375x over previous
import functools

import jax
import jax.numpy as jnp
from jax import lax
from jax.experimental import pallas as pl
from jax.experimental.pallas import tpu as pltpu
from jax.experimental.pallas import tpu_sc as plsc

_N = 10
_H = 512
_E = 160
_K = 3
_NP = 16


def _edge_counts_sc(edge_ref, out_ref, src_v, dst_v, idx_v, ones_v, stage_v,
                    out_v, acc):
    @pl.when((lax.axis_index("c") == 0) & (lax.axis_index("s") == 0))
    def _():
        pltpu.sync_copy(edge_ref.at[0], src_v)
        pltpu.sync_copy(edge_ref.at[1], dst_v)
        zrow = jnp.zeros((_NP,), jnp.float32)
        for r in range(_NP * _NP):
            stage_v[r, :] = zrow
        onerow = jnp.full((_NP,), 1.0, jnp.float32)
        for e in range(_E):
            ones_v[e, :] = onerow
        for c in range(_E // _NP):
            s_ids = src_v[pl.ds(c * _NP, _NP)]
            d_ids = dst_v[pl.ds(c * _NP, _NP)]
            idx_v[pl.ds(c * _NP, _NP)] = d_ids * _NP + s_ids
        pltpu.sync_copy(stage_v, acc)
        pltpu.sync_copy(ones_v, acc.at[idx_v], add=True)
        pltpu.sync_copy(acc, stage_v)
        lanes = lax.iota(jnp.int32, _NP)
        for d in range(_N):
            row = jnp.zeros((_NP,), jnp.float32)
            for s in range(_N):
                row = row + jnp.where(lanes == s, stage_v[d * _NP + s, :], 0.0)
            out_v[d, :] = row
        pltpu.sync_copy(out_v, out_ref)


_edge_counts = pl.kernel(
    _edge_counts_sc,
    mesh=plsc.VectorSubcoreMesh(core_axis_name="c", subcore_axis_name="s"),
    out_type=jax.ShapeDtypeStruct((_NP, _NP), jnp.float32),
    scratch_types=[pltpu.VMEM((_E,), jnp.int32),
                   pltpu.VMEM((_E,), jnp.int32),
                   pltpu.VMEM((_E,), jnp.int32),
                   pltpu.VMEM((_E, _NP), jnp.float32),
                   pltpu.VMEM((_NP * _NP, _NP), jnp.float32),
                   pltpu.VMEM((_NP, _NP), jnp.float32),
                   pltpu.VMEM_SHARED((_NP * _NP, _NP), jnp.float32)],
)


def _fused(x_ref, a_ref,
           b10, b20, b11, b21, b12, b22, b13, b23,
           sa_w1_ref, sa_w2_ref, sa_w3_ref, bs_ref, vs_wT_ref,
           fc_w_ref, fc_b_ref, gcn_w_ref, gcn_b_ref,
           w10, w20, w11, w21, w12, w22, w13, w23,
           bp_ref, bm_ref):
    f32 = jnp.float32
    w_refs = (w10, w20, w11, w21, w12, w22, w13, w23)
    b_refs = (b10, b20, b11, b21, b12, b22, b13, b23)

    X = x_ref[...]
    for i in range(8):
        out = jax.lax.dot_general(X, w_refs[i][...], (((1,), (1,)), ((), ())),
                                  preferred_element_type=f32)
        out = out + b_refs[i][...]
        if i % 2 == 0:
            X_res, X = X, jnp.maximum(out, 0.0)
        else:
            X = jnp.maximum(jnp.maximum(out, 0.0) + X_res, 0.0)
    emb = X

    left = jnp.sum(emb * sa_w2_ref[...], axis=1, keepdims=True) * sa_w1_ref[0, 0]
    right = jnp.sum(emb * sa_w3_ref[...], axis=1, keepdims=True)
    sa_x = jax.lax.dot_general(left, right, (((1,), (1,)), ((), ())),
                               preferred_element_type=f32)
    sa_x = sa_x + bs_ref[...]
    sa_s = jnp.dot(jax.nn.sigmoid(sa_x), vs_wT_ref[...],
                   preferred_element_type=f32)

    A = a_ref[...][:_N, :_N]
    deg_in = jnp.sum(A, axis=1, keepdims=True)
    deg_out = jnp.sum(A, axis=0, keepdims=True)
    norm_in = jnp.where(deg_in > 0,
                        jax.lax.rsqrt(jnp.maximum(deg_in, 1e-12)), 0.0)
    norm_out = jnp.where(deg_out > 0,
                         jax.lax.rsqrt(jnp.maximum(deg_out, 1e-12)), 0.0)
    M = A * norm_in * norm_out
    agg = jnp.dot(M, emb, preferred_element_type=f32)
    g_emb = jnp.dot(agg, gcn_w_ref[...], preferred_element_type=f32) + gcn_b_ref[...]

    sa_ag = jnp.dot(sa_s, g_emb, preferred_element_type=f32)
    logits = jnp.sum(sa_ag * fc_w_ref[...], axis=1, keepdims=True) + fc_b_ref[0, 0]
    scores = jax.nn.sigmoid(logits)

    e1 = jnp.exp(scores)
    bp_ref[...] = e1 / jnp.sum(e1)
    e2 = jnp.exp(1.0 - scores)
    bm_ref[...] = e2 / jnp.sum(e2)


@jax.jit
def kernel(x, edge_index, tcn_params, sa_w1, sa_w2, sa_w3, bs, vs_w,
           fc_w, fc_b, gcn_w, gcn_b):
    A16 = _edge_counts(edge_index)

    vmem_ins = [x[:, :, 0], A16]
    vmem_ins += [b[None, :] for (w1, b1, w2, b2) in tcn_params for b in (b1, b2)]
    vmem_ins += [sa_w1, sa_w2.T, sa_w3, bs[None, :], vs_w.T,
                 fc_w, fc_b[None, :], gcn_w, gcn_b[None, :]]
    w_ins = [w[:, :, _K - 1] for (w1, b1, w2, b2) in tcn_params
             for w in (w1, w2)]

    bp, bm = pl.pallas_call(
        _fused,
        out_shape=[jax.ShapeDtypeStruct((_N, 1), jnp.float32),
                   jax.ShapeDtypeStruct((_N, 1), jnp.float32)],
    )(*vmem_ins, *w_ins)
    return bp[:, 0], bm[:, 0]
